# baseline (speedup 1.0000x reference)
import jax
import jax.numpy as jnp
from jax.experimental import pallas as pl
from jax.experimental.pallas import tpu as pltpu

K_DPC = 5
TOPK = 5
TN = 1792

HIGH = jax.lax.Precision.HIGHEST


def _dpc_kernel(tok_ref, rp_ref, cent_ref):
    tok = tok_ref[...]
    npts = tok.shape[1]
    aa = jnp.transpose(jnp.sum(tok * tok, axis=0, keepdims=True))
    ab = jax.lax.dot_general(tok, tok, (((0,), (0,)), ((), ())),
                             precision=HIGH)
    d2 = jnp.maximum(aa + jnp.transpose(aa) - 2.0 * ab, 0.0)
    dist = jnp.sqrt(d2 + 1e-12) + rp_ref[...]
    colid = jax.lax.broadcasted_iota(jnp.int32, (npts, npts), 1)
    work = dist
    acc = jnp.zeros((npts, 1), jnp.float32)
    for _ in range(K_DPC):
        mn = jnp.min(work, axis=1, keepdims=True)
        sel = jnp.min(jnp.where(work == mn, colid, npts), axis=1, keepdims=True)
        acc = acc + mn * mn
        work = jnp.where(colid == sel, jnp.inf, work)
    density = jnp.exp(-(acc / jnp.float32(K_DPC)))
    higher = jnp.transpose(density) > density
    dist_max = jnp.max(dist)
    delta = jnp.min(jnp.where(higher, dist, dist_max), axis=1, keepdims=True)
    score = delta * density
    st = jnp.transpose(score)
    rowid = jax.lax.broadcasted_iota(jnp.int32, (npts, npts), 0)
    rank = (jnp.sum((st > score).astype(jnp.int32), axis=1, keepdims=True)
            + jnp.sum(((st == score) & (colid < rowid)).astype(jnp.int32),
                      axis=1, keepdims=True))
    m = cent_ref.shape[0]
    sel_mat = (jax.lax.broadcasted_iota(jnp.int32, (m, npts), 0)
               == jnp.transpose(rank)).astype(jnp.float32)
    cent_ref[...] = jax.lax.dot_general(sel_mat, tok, (((1,), (1,)), ((), ())),
                                        precision=HIGH)


def _pass1_kernel(cent_ref, x_ref, idx_ref, num_ref, den_ref):
    i = pl.program_id(0)
    cent = cent_ref[...]
    xt = x_ref[...]
    m, c = cent.shape
    sim = jax.lax.dot_general(cent, xt, (((1,), (0,)), ((), ())),
                              precision=HIGH) / jnp.sqrt(jnp.float32(c))
    e = jnp.exp(sim - jnp.max(sim, axis=0, keepdims=True))
    assign = e / jnp.sum(e, axis=0, keepdims=True)

    @pl.when(i == 0)
    def _():
        num_ref[...] = jnp.zeros_like(num_ref)
        den_ref[...] = jnp.zeros_like(den_ref)

    num_ref[...] += jax.lax.dot_general(assign, xt, (((1,), (1,)), ((), ())),
                                        precision=HIGH)
    den_ref[...] += jnp.sum(assign, axis=1, keepdims=True)

    rowid = jax.lax.broadcasted_iota(jnp.int32, assign.shape, 0)
    work = assign
    rows = []
    for _ in range(TOPK):
        mx = jnp.max(work, axis=0, keepdims=True)
        sel = jnp.min(jnp.where(work == mx, rowid, m), axis=0, keepdims=True)
        rows.append(sel)
        work = jnp.where(rowid == sel, -jnp.inf, work)
    pad = jnp.zeros_like(rows[0])
    idx_ref[...] = jnp.concatenate(rows + [pad, pad, pad], axis=0)


def _prep_kernel(num_ref, den_ref, W1_ref, b1_ref, W2_ref, b2_ref, Wnn_ref,
                 refined_ref, rhi_ref, rlo_ref, Wa_ref, Wb_ref):
    agg = num_ref[...] / (den_ref[...] + 1e-6)
    h1 = jax.lax.dot_general(agg, W1_ref[...], (((1,), (1,)), ((), ())),
                             precision=HIGH) + b1_ref[...]
    h1 = jnp.maximum(h1, 0.0)
    ffn = jax.lax.dot_general(h1, W2_ref[...], (((1,), (1,)), ((), ())),
                              precision=HIGH) + b2_ref[...]
    refined = agg + ffn
    refined_ref[...] = jnp.transpose(refined)
    rhi = refined.astype(jnp.bfloat16)
    rhi_ref[...] = rhi
    rlo_ref[...] = (refined - rhi.astype(jnp.float32)).astype(jnp.bfloat16)
    Wnn = Wnn_ref[...]
    c = Wnn.shape[0]
    r2 = jax.lax.broadcasted_iota(jnp.int32, (2 * c, c), 0)
    c2 = jax.lax.broadcasted_iota(jnp.int32, (2 * c, c), 1)
    sa = (r2 == 2 * c2).astype(jnp.float32)
    sb = (r2 == 2 * c2 + 1).astype(jnp.float32)
    Wa_ref[...] = jax.lax.dot_general(Wnn, sa, (((1,), (0,)), ((), ())),
                                      precision=HIGH)
    Wb_ref[...] = jax.lax.dot_general(Wnn, sb, (((1,), (0,)), ((), ())),
                                      precision=HIGH)


def _pass2_kernel(x_ref, idx_ref, rhi_ref, rlo_ref, Wa_ref, Wb_ref, bnn_ref,
                  z_ref, stats_ref, s_ref, s2_ref):
    i = pl.program_id(0)
    xt = x_ref[...]
    rhi = rhi_ref[...]
    rlo = rlo_ref[...]
    m = rhi.shape[0]
    idx = idx_ref[...]
    rowid = jax.lax.broadcasted_iota(jnp.int32, (m, xt.shape[1]), 0)
    mx = None
    for k in range(TOPK):
        hot = (rowid == idx[k:k + 1, :]).astype(jnp.bfloat16)
        g = jax.lax.dot_general(
            rhi, hot, (((0,), (0,)), ((), ())),
            preferred_element_type=jnp.float32)
        g = g + jax.lax.dot_general(
            rlo, hot, (((0,), (0,)), ((), ())),
            preferred_element_type=jnp.float32)
        mx = g if mx is None else jnp.maximum(mx, g)
    xj = mx - xt
    z = (jax.lax.dot_general(Wa_ref[...], xt, (((1,), (0,)), ((), ())),
                             precision=HIGH)
         + jax.lax.dot_general(Wb_ref[...], xj, (((1,), (0,)), ((), ())),
                               precision=HIGH)
         + bnn_ref[...])
    z_ref[...] = z

    @pl.when(i == 0)
    def _():
        s_ref[...] = jnp.zeros_like(s_ref)
        s2_ref[...] = jnp.zeros_like(s2_ref)

    s_ref[...] += jnp.sum(z, axis=1, keepdims=True)
    s2_ref[...] += jnp.sum(z * z, axis=1, keepdims=True)

    @pl.when(i == pl.num_programs(0) - 1)
    def _():
        stats_ref[...] = jnp.concatenate(
            [s_ref[...], s2_ref[...],
             jnp.zeros((s_ref.shape[0], 6), jnp.float32)], axis=1)


def _bn_kernel(z_ref, stats_ref, gamma_ref, beta_ref, out_ref):
    n = jnp.float32(z_ref.shape[1] * pl.num_programs(0))
    mu = stats_ref[:, 0:1] / n
    var = stats_ref[:, 1:2] / n - mu * mu
    inv = gamma_ref[...] / jnp.sqrt(var + 1e-5)
    out_ref[...] = jnp.maximum((z_ref[...] - mu) * inv + beta_ref[...], 0.0)


def kernel(x, relative_pos, y, W1, b1, W2, b2, Wnn, bnn, gamma, beta):
    b, c, h, w = x.shape
    n = h * w
    x_flat = x.reshape(c, n)
    npts = y.shape[2]
    m = int(0.25 * npts)
    tok = y[0, :, :, 0]

    centroids = pl.pallas_call(
        _dpc_kernel,
        out_shape=jax.ShapeDtypeStruct((m, c), jnp.float32),
    )(tok, relative_pos[0])

    nsteps = n // TN
    idx, num, den = pl.pallas_call(
        _pass1_kernel,
        grid=(nsteps,),
        in_specs=[
            pl.BlockSpec((m, c), lambda i: (0, 0)),
            pl.BlockSpec((c, TN), lambda i: (0, i)),
        ],
        out_specs=[
            pl.BlockSpec((8, TN), lambda i: (0, i)),
            pl.BlockSpec((m, c), lambda i: (0, 0)),
            pl.BlockSpec((m, 1), lambda i: (0, 0)),
        ],
        out_shape=[
            jax.ShapeDtypeStruct((8, n), jnp.int32),
            jax.ShapeDtypeStruct((m, c), jnp.float32),
            jax.ShapeDtypeStruct((m, 1), jnp.float32),
        ],
        compiler_params=pltpu.CompilerParams(
            dimension_semantics=("arbitrary",)),
    )(centroids, x_flat)

    refined_cm, rhi, rlo, Wa, Wb = pl.pallas_call(
        _prep_kernel,
        out_shape=[
            jax.ShapeDtypeStruct((c, m), jnp.float32),
            jax.ShapeDtypeStruct((m, c), jnp.bfloat16),
            jax.ShapeDtypeStruct((m, c), jnp.bfloat16),
            jax.ShapeDtypeStruct((c, c), jnp.float32),
            jax.ShapeDtypeStruct((c, c), jnp.float32),
        ],
    )(num, den, W1, b1.reshape(1, -1), W2, b2.reshape(1, -1), Wnn)

    z, stats = pl.pallas_call(
        _pass2_kernel,
        grid=(nsteps,),
        in_specs=[
            pl.BlockSpec((c, TN), lambda i: (0, i)),
            pl.BlockSpec((8, TN), lambda i: (0, i)),
            pl.BlockSpec((m, c), lambda i: (0, 0)),
            pl.BlockSpec((m, c), lambda i: (0, 0)),
            pl.BlockSpec((c, c), lambda i: (0, 0)),
            pl.BlockSpec((c, c), lambda i: (0, 0)),
            pl.BlockSpec((c, 1), lambda i: (0, 0)),
        ],
        out_specs=[
            pl.BlockSpec((c, TN), lambda i: (0, i)),
            pl.BlockSpec((c, 8), lambda i: (0, 0)),
        ],
        out_shape=[
            jax.ShapeDtypeStruct((c, n), jnp.float32),
            jax.ShapeDtypeStruct((c, 8), jnp.float32),
        ],
        scratch_shapes=[
            pltpu.VMEM((c, 1), jnp.float32),
            pltpu.VMEM((c, 1), jnp.float32),
        ],
        compiler_params=pltpu.CompilerParams(
            dimension_semantics=("arbitrary",)),
    )(x_flat, idx, rhi, rlo, Wa, Wb, bnn.reshape(-1, 1))

    out2d = pl.pallas_call(
        _bn_kernel,
        grid=(nsteps,),
        in_specs=[
            pl.BlockSpec((c, TN), lambda i: (0, i)),
            pl.BlockSpec((c, 8), lambda i: (0, 0)),
            pl.BlockSpec((c, 1), lambda i: (0, 0)),
            pl.BlockSpec((c, 1), lambda i: (0, 0)),
        ],
        out_specs=pl.BlockSpec((c, TN), lambda i: (0, i)),
        out_shape=jax.ShapeDtypeStruct((c, n), jnp.float32),
        compiler_params=pltpu.CompilerParams(
            dimension_semantics=("arbitrary",)),
    )(z, stats, gamma.reshape(-1, 1), beta.reshape(-1, 1))

    out = out2d.reshape(1, c, n, 1)
    refined = refined_cm.reshape(1, c, m, 1)
    return out, refined


# speedup vs baseline: 1.0007x; 1.0007x over previous
import jax
import jax.numpy as jnp
from jax.experimental import pallas as pl
from jax.experimental.pallas import tpu as pltpu

K_DPC = 5
TOPK = 5
TN = 1792

HIGH = jax.lax.Precision.HIGHEST


def _dpc_kernel(tok_ref, rp_ref, cent_ref):
    tok = jnp.transpose(tok_ref[...])
    npts = tok.shape[0]
    aa = jnp.sum(tok * tok, axis=1, keepdims=True)
    ab = jax.lax.dot_general(tok, tok, (((1,), (1,)), ((), ())),
                             precision=HIGH)
    d2 = jnp.maximum(aa + jnp.transpose(aa) - 2.0 * ab, 0.0)
    dist = jnp.sqrt(d2 + 1e-12) + rp_ref[...]
    colid = jax.lax.broadcasted_iota(jnp.int32, (npts, npts), 1)
    work = dist
    acc = jnp.zeros((npts, 1), jnp.float32)
    for _ in range(K_DPC):
        mn = jnp.min(work, axis=1, keepdims=True)
        sel = jnp.min(jnp.where(work == mn, colid, npts), axis=1, keepdims=True)
        acc = acc + mn * mn
        work = jnp.where(colid == sel, jnp.inf, work)
    density = jnp.exp(-(acc / jnp.float32(K_DPC)))
    higher = jnp.transpose(density) > density
    dist_max = jnp.max(dist)
    delta = jnp.min(jnp.where(higher, dist, dist_max), axis=1, keepdims=True)
    score = delta * density
    st = jnp.transpose(score)
    rowid = jax.lax.broadcasted_iota(jnp.int32, (npts, npts), 0)
    rank = (jnp.sum((st > score).astype(jnp.int32), axis=1, keepdims=True)
            + jnp.sum(((st == score) & (colid < rowid)).astype(jnp.int32),
                      axis=1, keepdims=True))
    m = cent_ref.shape[0]
    sel_mat = (jax.lax.broadcasted_iota(jnp.int32, (m, npts), 0)
               == jnp.transpose(rank)).astype(jnp.float32)
    cent_ref[...] = jax.lax.dot_general(sel_mat, tok, (((1,), (0,)), ((), ())),
                                        precision=HIGH)


def _pass1_kernel(cent_ref, x_ref, idx_ref, num_ref, den_ref):
    i = pl.program_id(0)
    cent = cent_ref[...]
    xt = x_ref[...]
    m, c = cent.shape
    sim = jax.lax.dot_general(cent, xt, (((1,), (0,)), ((), ())),
                              precision=HIGH) / jnp.sqrt(jnp.float32(c))
    e = jnp.exp(sim - jnp.max(sim, axis=0, keepdims=True))
    assign = e / jnp.sum(e, axis=0, keepdims=True)

    @pl.when(i == 0)
    def _():
        num_ref[...] = jnp.zeros_like(num_ref)
        den_ref[...] = jnp.zeros_like(den_ref)

    num_ref[...] += jax.lax.dot_general(assign, xt, (((1,), (1,)), ((), ())),
                                        precision=HIGH)
    den_ref[...] += jnp.sum(assign, axis=1, keepdims=True)

    rowid = jax.lax.broadcasted_iota(jnp.int32, assign.shape, 0)
    work = assign
    rows = []
    for _ in range(TOPK):
        mx = jnp.max(work, axis=0, keepdims=True)
        sel = jnp.min(jnp.where(work == mx, rowid, m), axis=0, keepdims=True)
        rows.append(sel)
        work = jnp.where(rowid == sel, -jnp.inf, work)
    pad = jnp.zeros_like(rows[0])
    idx_ref[...] = jnp.concatenate(rows + [pad, pad, pad], axis=0)


def _prep_kernel(num_ref, den_ref, W1_ref, b1_ref, W2_ref, b2_ref, Wnn_ref,
                 refined_ref, rhi_ref, rlo_ref, Wa_ref, Wb_ref):
    agg = num_ref[...] / (den_ref[...] + 1e-6)
    h1 = jax.lax.dot_general(agg, W1_ref[...], (((1,), (1,)), ((), ())),
                             precision=HIGH) + b1_ref[...]
    h1 = jnp.maximum(h1, 0.0)
    ffn = jax.lax.dot_general(h1, W2_ref[...], (((1,), (1,)), ((), ())),
                              precision=HIGH) + b2_ref[...]
    refined = agg + ffn
    m = refined.shape[0]
    eye_m = (jax.lax.broadcasted_iota(jnp.int32, (m, m), 0)
             == jax.lax.broadcasted_iota(jnp.int32, (m, m), 1)).astype(jnp.float32)
    refined_ref[...] = jax.lax.dot_general(refined, eye_m, (((0,), (0,)), ((), ())),
                                           precision=HIGH)
    rhi = refined.astype(jnp.bfloat16)
    rhi_ref[...] = rhi
    rlo_ref[...] = (refined - rhi.astype(jnp.float32)).astype(jnp.bfloat16)
    Wnn = Wnn_ref[...]
    c = Wnn.shape[0]
    r2 = jax.lax.broadcasted_iota(jnp.int32, (2 * c, c), 0)
    c2 = jax.lax.broadcasted_iota(jnp.int32, (2 * c, c), 1)
    sa = (r2 == 2 * c2).astype(jnp.float32)
    sb = (r2 == 2 * c2 + 1).astype(jnp.float32)
    Wa_ref[...] = jax.lax.dot_general(Wnn, sa, (((1,), (0,)), ((), ())),
                                      precision=HIGH)
    Wb_ref[...] = jax.lax.dot_general(Wnn, sb, (((1,), (0,)), ((), ())),
                                      precision=HIGH)


def _pass2_kernel(x_ref, idx_ref, rhi_ref, rlo_ref, Wa_ref, Wb_ref, bnn_ref,
                  z_ref, stats_ref, s_ref, s2_ref):
    i = pl.program_id(0)
    xt = x_ref[...]
    rhi = rhi_ref[...]
    rlo = rlo_ref[...]
    m = rhi.shape[0]
    idx = idx_ref[...]
    rowid = jax.lax.broadcasted_iota(jnp.int32, (m, xt.shape[1]), 0)
    mx = None
    for k in range(TOPK):
        hot = (rowid == idx[k:k + 1, :]).astype(jnp.bfloat16)
        g = jax.lax.dot_general(
            rhi, hot, (((0,), (0,)), ((), ())),
            preferred_element_type=jnp.float32)
        g = g + jax.lax.dot_general(
            rlo, hot, (((0,), (0,)), ((), ())),
            preferred_element_type=jnp.float32)
        mx = g if mx is None else jnp.maximum(mx, g)
    xj = mx - xt
    z = (jax.lax.dot_general(Wa_ref[...], xt, (((1,), (0,)), ((), ())),
                             precision=HIGH)
         + jax.lax.dot_general(Wb_ref[...], xj, (((1,), (0,)), ((), ())),
                               precision=HIGH)
         + bnn_ref[...])
    z_ref[...] = z

    @pl.when(i == 0)
    def _():
        s_ref[...] = jnp.zeros_like(s_ref)
        s2_ref[...] = jnp.zeros_like(s2_ref)

    s_ref[...] += jnp.sum(z, axis=1, keepdims=True)
    s2_ref[...] += jnp.sum(z * z, axis=1, keepdims=True)

    @pl.when(i == pl.num_programs(0) - 1)
    def _():
        stats_ref[...] = jnp.concatenate(
            [s_ref[...], s2_ref[...],
             jnp.zeros((s_ref.shape[0], 6), jnp.float32)], axis=1)


def _bn_kernel(z_ref, stats_ref, gamma_ref, beta_ref, out_ref):
    n = jnp.float32(z_ref.shape[1] * pl.num_programs(0))
    mu = stats_ref[:, 0:1] / n
    var = stats_ref[:, 1:2] / n - mu * mu
    inv = gamma_ref[...] / jnp.sqrt(var + 1e-5)
    out_ref[...] = jnp.maximum((z_ref[...] - mu) * inv + beta_ref[...], 0.0)


def kernel(x, relative_pos, y, W1, b1, W2, b2, Wnn, bnn, gamma, beta):
    b, c, h, w = x.shape
    n = h * w
    x_flat = x.reshape(c, n)
    npts = y.shape[2]
    m = int(0.25 * npts)
    tok = y[0, :, :, 0]

    centroids = pl.pallas_call(
        _dpc_kernel,
        out_shape=jax.ShapeDtypeStruct((m, c), jnp.float32),
    )(tok, relative_pos[0])

    nsteps = n // TN
    idx, num, den = pl.pallas_call(
        _pass1_kernel,
        grid=(nsteps,),
        in_specs=[
            pl.BlockSpec((m, c), lambda i: (0, 0)),
            pl.BlockSpec((c, TN), lambda i: (0, i)),
        ],
        out_specs=[
            pl.BlockSpec((8, TN), lambda i: (0, i)),
            pl.BlockSpec((m, c), lambda i: (0, 0)),
            pl.BlockSpec((m, 1), lambda i: (0, 0)),
        ],
        out_shape=[
            jax.ShapeDtypeStruct((8, n), jnp.int32),
            jax.ShapeDtypeStruct((m, c), jnp.float32),
            jax.ShapeDtypeStruct((m, 1), jnp.float32),
        ],
        compiler_params=pltpu.CompilerParams(
            dimension_semantics=("arbitrary",)),
    )(centroids, x_flat)

    refined_cm, rhi, rlo, Wa, Wb = pl.pallas_call(
        _prep_kernel,
        out_shape=[
            jax.ShapeDtypeStruct((c, m), jnp.float32),
            jax.ShapeDtypeStruct((m, c), jnp.bfloat16),
            jax.ShapeDtypeStruct((m, c), jnp.bfloat16),
            jax.ShapeDtypeStruct((c, c), jnp.float32),
            jax.ShapeDtypeStruct((c, c), jnp.float32),
        ],
    )(num, den, W1, b1.reshape(1, -1), W2, b2.reshape(1, -1), Wnn)

    z, stats = pl.pallas_call(
        _pass2_kernel,
        grid=(nsteps,),
        in_specs=[
            pl.BlockSpec((c, TN), lambda i: (0, i)),
            pl.BlockSpec((8, TN), lambda i: (0, i)),
            pl.BlockSpec((m, c), lambda i: (0, 0)),
            pl.BlockSpec((m, c), lambda i: (0, 0)),
            pl.BlockSpec((c, c), lambda i: (0, 0)),
            pl.BlockSpec((c, c), lambda i: (0, 0)),
            pl.BlockSpec((c, 1), lambda i: (0, 0)),
        ],
        out_specs=[
            pl.BlockSpec((c, TN), lambda i: (0, i)),
            pl.BlockSpec((c, 8), lambda i: (0, 0)),
        ],
        out_shape=[
            jax.ShapeDtypeStruct((c, n), jnp.float32),
            jax.ShapeDtypeStruct((c, 8), jnp.float32),
        ],
        scratch_shapes=[
            pltpu.VMEM((c, 1), jnp.float32),
            pltpu.VMEM((c, 1), jnp.float32),
        ],
        compiler_params=pltpu.CompilerParams(
            dimension_semantics=("arbitrary",)),
    )(x_flat, idx, rhi, rlo, Wa, Wb, bnn.reshape(-1, 1))

    out2d = pl.pallas_call(
        _bn_kernel,
        grid=(nsteps,),
        in_specs=[
            pl.BlockSpec((c, TN), lambda i: (0, i)),
            pl.BlockSpec((c, 8), lambda i: (0, 0)),
            pl.BlockSpec((c, 1), lambda i: (0, 0)),
            pl.BlockSpec((c, 1), lambda i: (0, 0)),
        ],
        out_specs=pl.BlockSpec((c, TN), lambda i: (0, i)),
        out_shape=jax.ShapeDtypeStruct((c, n), jnp.float32),
        compiler_params=pltpu.CompilerParams(
            dimension_semantics=("arbitrary",)),
    )(z, stats, gamma.reshape(-1, 1), beta.reshape(-1, 1))

    out = out2d.reshape(1, c, n, 1)
    refined = refined_cm.reshape(1, c, m, 1)
    return out, refined


# speedup vs baseline: 1.1316x; 1.1309x over previous
import jax
import jax.numpy as jnp
from jax.experimental import pallas as pl
from jax.experimental.pallas import tpu as pltpu

K_DPC = 5
TOPK = 5
TN = 1792

HIGH = jax.lax.Precision.HIGHEST


def _split(a):
    hi = a.astype(jnp.bfloat16)
    lo = (a - hi.astype(jnp.float32)).astype(jnp.bfloat16)
    return hi, lo


def _dot3(a, b, dims):
    ahi, alo = _split(a)
    bhi, blo = _split(b)
    d = lambda u, v: jax.lax.dot_general(u, v, dims,
                                         preferred_element_type=jnp.float32)
    return d(ahi, bhi) + (d(ahi, blo) + d(alo, bhi))


def _dpc_kernel(tok_ref, rp_ref, cent_ref):
    tok = jnp.transpose(tok_ref[...])
    npts = tok.shape[0]
    aa = jnp.sum(tok * tok, axis=1, keepdims=True)
    ab = jax.lax.dot_general(tok, tok, (((1,), (1,)), ((), ())),
                             precision=HIGH)
    d2 = jnp.maximum(aa + jnp.transpose(aa) - 2.0 * ab, 0.0)
    dist = jnp.sqrt(d2 + 1e-12) + rp_ref[...]
    colid = jax.lax.broadcasted_iota(jnp.int32, (npts, npts), 1)
    work = dist
    acc = jnp.zeros((npts, 1), jnp.float32)
    for _ in range(K_DPC):
        mn = jnp.min(work, axis=1, keepdims=True)
        sel = jnp.min(jnp.where(work == mn, colid, npts), axis=1, keepdims=True)
        acc = acc + mn * mn
        work = jnp.where(colid == sel, jnp.inf, work)
    density = jnp.exp(-(acc / jnp.float32(K_DPC)))
    higher = jnp.transpose(density) > density
    dist_max = jnp.max(dist)
    delta = jnp.min(jnp.where(higher, dist, dist_max), axis=1, keepdims=True)
    score = delta * density
    st = jnp.transpose(score)
    rowid = jax.lax.broadcasted_iota(jnp.int32, (npts, npts), 0)
    rank = (jnp.sum((st > score).astype(jnp.int32), axis=1, keepdims=True)
            + jnp.sum(((st == score) & (colid < rowid)).astype(jnp.int32),
                      axis=1, keepdims=True))
    m = cent_ref.shape[0]
    sel_mat = (jax.lax.broadcasted_iota(jnp.int32, (m, npts), 0)
               == jnp.transpose(rank)).astype(jnp.float32)
    cent_ref[...] = jax.lax.dot_general(sel_mat, tok, (((1,), (0,)), ((), ())),
                                        precision=HIGH)


def _pass1_kernel(cent_ref, x_ref, idx_ref, num_ref, den_ref):
    i = pl.program_id(0)
    cent = cent_ref[...]
    xt = x_ref[...]
    m, c = cent.shape
    sim = jax.lax.dot_general(cent, xt, (((1,), (0,)), ((), ())),
                              precision=HIGH) / jnp.sqrt(jnp.float32(c))
    e = jnp.exp(sim - jnp.max(sim, axis=0, keepdims=True))
    assign = e / jnp.sum(e, axis=0, keepdims=True)

    @pl.when(i == 0)
    def _():
        num_ref[...] = jnp.zeros_like(num_ref)
        den_ref[...] = jnp.zeros_like(den_ref)

    num_ref[...] += _dot3(assign, xt, (((1,), (1,)), ((), ())))
    den_ref[...] += jnp.sum(assign, axis=1, keepdims=True)

    rowid = jax.lax.broadcasted_iota(jnp.int32, assign.shape, 0)
    work = assign
    rows = []
    for _ in range(TOPK):
        mx = jnp.max(work, axis=0, keepdims=True)
        sel = jnp.min(jnp.where(work == mx, rowid, m), axis=0, keepdims=True)
        rows.append(sel)
        work = jnp.where(rowid == sel, -jnp.inf, work)
    pad = jnp.zeros_like(rows[0])
    idx_ref[...] = jnp.concatenate(rows + [pad, pad, pad], axis=0)


def _prep_kernel(num_ref, den_ref, W1_ref, b1_ref, W2_ref, b2_ref, Wnn_ref,
                 refined_ref, rhi_ref, rlo_ref,
                 Wahi_ref, Walo_ref, Wbhi_ref, Wblo_ref):
    agg = num_ref[...] / (den_ref[...] + 1e-6)
    h1 = jax.lax.dot_general(agg, W1_ref[...], (((1,), (1,)), ((), ())),
                             precision=HIGH) + b1_ref[...]
    h1 = jnp.maximum(h1, 0.0)
    ffn = jax.lax.dot_general(h1, W2_ref[...], (((1,), (1,)), ((), ())),
                              precision=HIGH) + b2_ref[...]
    refined = agg + ffn
    m = refined.shape[0]
    eye_m = (jax.lax.broadcasted_iota(jnp.int32, (m, m), 0)
             == jax.lax.broadcasted_iota(jnp.int32, (m, m), 1)).astype(jnp.float32)
    refined_ref[...] = jax.lax.dot_general(refined, eye_m, (((0,), (0,)), ((), ())),
                                           precision=HIGH)
    rhi = refined.astype(jnp.bfloat16)
    rhi_ref[...] = rhi
    rlo_ref[...] = (refined - rhi.astype(jnp.float32)).astype(jnp.bfloat16)
    Wnn = Wnn_ref[...]
    c = Wnn.shape[0]
    r2 = jax.lax.broadcasted_iota(jnp.int32, (2 * c, c), 0)
    c2 = jax.lax.broadcasted_iota(jnp.int32, (2 * c, c), 1)
    sa = (r2 == 2 * c2).astype(jnp.float32)
    sb = (r2 == 2 * c2 + 1).astype(jnp.float32)
    Wa = jax.lax.dot_general(Wnn, sa, (((1,), (0,)), ((), ())),
                             precision=HIGH)
    Wb = jax.lax.dot_general(Wnn, sb, (((1,), (0,)), ((), ())),
                             precision=HIGH)
    Wahi_ref[...], Walo_ref[...] = _split(Wa)
    Wbhi_ref[...], Wblo_ref[...] = _split(Wb)


def _pass2_kernel(x_ref, idx_ref, rhi_ref, rlo_ref,
                  Wahi_ref, Walo_ref, Wbhi_ref, Wblo_ref, bnn_ref,
                  z_ref, stats_ref, s_ref, s2_ref):
    i = pl.program_id(0)
    xt = x_ref[...]
    rhi = rhi_ref[...]
    rlo = rlo_ref[...]
    m = rhi.shape[0]
    idx = idx_ref[...]
    rowid = jax.lax.broadcasted_iota(jnp.int32, (m, xt.shape[1]), 0)
    mx = None
    for k in range(TOPK):
        hot = (rowid == idx[k:k + 1, :]).astype(jnp.bfloat16)
        g = jax.lax.dot_general(
            rhi, hot, (((0,), (0,)), ((), ())),
            preferred_element_type=jnp.float32)
        g = g + jax.lax.dot_general(
            rlo, hot, (((0,), (0,)), ((), ())),
            preferred_element_type=jnp.float32)
        mx = g if mx is None else jnp.maximum(mx, g)
    xj = mx - xt
    dims = (((1,), (0,)), ((), ()))
    d = lambda u, v: jax.lax.dot_general(u, v, dims,
                                         preferred_element_type=jnp.float32)
    xthi, xtlo = _split(xt)
    xjhi, xjlo = _split(xj)
    z = (d(Wahi_ref[...], xthi) + (d(Wahi_ref[...], xtlo)
                                   + d(Walo_ref[...], xthi))
         + d(Wbhi_ref[...], xjhi) + (d(Wbhi_ref[...], xjlo)
                                     + d(Wblo_ref[...], xjhi))
         + bnn_ref[...])
    z_ref[...] = z

    @pl.when(i == 0)
    def _():
        s_ref[...] = jnp.zeros_like(s_ref)
        s2_ref[...] = jnp.zeros_like(s2_ref)

    s_ref[...] += jnp.sum(z, axis=1, keepdims=True)
    s2_ref[...] += jnp.sum(z * z, axis=1, keepdims=True)

    @pl.when(i == pl.num_programs(0) - 1)
    def _():
        stats_ref[...] = jnp.concatenate(
            [s_ref[...], s2_ref[...],
             jnp.zeros((s_ref.shape[0], 6), jnp.float32)], axis=1)


def _bn_kernel(z_ref, stats_ref, gamma_ref, beta_ref, out_ref):
    n = jnp.float32(z_ref.shape[1] * pl.num_programs(0))
    mu = stats_ref[:, 0:1] / n
    var = stats_ref[:, 1:2] / n - mu * mu
    inv = gamma_ref[...] / jnp.sqrt(var + 1e-5)
    out_ref[...] = jnp.maximum((z_ref[...] - mu) * inv + beta_ref[...], 0.0)


def kernel(x, relative_pos, y, W1, b1, W2, b2, Wnn, bnn, gamma, beta):
    b, c, h, w = x.shape
    n = h * w
    x_flat = x.reshape(c, n)
    npts = y.shape[2]
    m = int(0.25 * npts)
    tok = y[0, :, :, 0]

    centroids = pl.pallas_call(
        _dpc_kernel,
        out_shape=jax.ShapeDtypeStruct((m, c), jnp.float32),
    )(tok, relative_pos[0])

    nsteps = n // TN
    idx, num, den = pl.pallas_call(
        _pass1_kernel,
        grid=(nsteps,),
        in_specs=[
            pl.BlockSpec((m, c), lambda i: (0, 0)),
            pl.BlockSpec((c, TN), lambda i: (0, i)),
        ],
        out_specs=[
            pl.BlockSpec((8, TN), lambda i: (0, i)),
            pl.BlockSpec((m, c), lambda i: (0, 0)),
            pl.BlockSpec((m, 1), lambda i: (0, 0)),
        ],
        out_shape=[
            jax.ShapeDtypeStruct((8, n), jnp.int32),
            jax.ShapeDtypeStruct((m, c), jnp.float32),
            jax.ShapeDtypeStruct((m, 1), jnp.float32),
        ],
        compiler_params=pltpu.CompilerParams(
            dimension_semantics=("arbitrary",)),
    )(centroids, x_flat)

    refined_cm, rhi, rlo, Wahi, Walo, Wbhi, Wblo = pl.pallas_call(
        _prep_kernel,
        out_shape=[
            jax.ShapeDtypeStruct((c, m), jnp.float32),
            jax.ShapeDtypeStruct((m, c), jnp.bfloat16),
            jax.ShapeDtypeStruct((m, c), jnp.bfloat16),
            jax.ShapeDtypeStruct((c, c), jnp.bfloat16),
            jax.ShapeDtypeStruct((c, c), jnp.bfloat16),
            jax.ShapeDtypeStruct((c, c), jnp.bfloat16),
            jax.ShapeDtypeStruct((c, c), jnp.bfloat16),
        ],
    )(num, den, W1, b1.reshape(1, -1), W2, b2.reshape(1, -1), Wnn)

    z, stats = pl.pallas_call(
        _pass2_kernel,
        grid=(nsteps,),
        in_specs=[
            pl.BlockSpec((c, TN), lambda i: (0, i)),
            pl.BlockSpec((8, TN), lambda i: (0, i)),
            pl.BlockSpec((m, c), lambda i: (0, 0)),
            pl.BlockSpec((m, c), lambda i: (0, 0)),
            pl.BlockSpec((c, c), lambda i: (0, 0)),
            pl.BlockSpec((c, c), lambda i: (0, 0)),
            pl.BlockSpec((c, c), lambda i: (0, 0)),
            pl.BlockSpec((c, c), lambda i: (0, 0)),
            pl.BlockSpec((c, 1), lambda i: (0, 0)),
        ],
        out_specs=[
            pl.BlockSpec((c, TN), lambda i: (0, i)),
            pl.BlockSpec((c, 8), lambda i: (0, 0)),
        ],
        out_shape=[
            jax.ShapeDtypeStruct((c, n), jnp.float32),
            jax.ShapeDtypeStruct((c, 8), jnp.float32),
        ],
        scratch_shapes=[
            pltpu.VMEM((c, 1), jnp.float32),
            pltpu.VMEM((c, 1), jnp.float32),
        ],
        compiler_params=pltpu.CompilerParams(
            dimension_semantics=("arbitrary",)),
    )(x_flat, idx, rhi, rlo, Wahi, Walo, Wbhi, Wblo, bnn.reshape(-1, 1))

    out2d = pl.pallas_call(
        _bn_kernel,
        grid=(nsteps,),
        in_specs=[
            pl.BlockSpec((c, TN), lambda i: (0, i)),
            pl.BlockSpec((c, 8), lambda i: (0, 0)),
            pl.BlockSpec((c, 1), lambda i: (0, 0)),
            pl.BlockSpec((c, 1), lambda i: (0, 0)),
        ],
        out_specs=pl.BlockSpec((c, TN), lambda i: (0, i)),
        out_shape=jax.ShapeDtypeStruct((c, n), jnp.float32),
        compiler_params=pltpu.CompilerParams(
            dimension_semantics=("arbitrary",)),
    )(z, stats, gamma.reshape(-1, 1), beta.reshape(-1, 1))

    out = out2d.reshape(1, c, n, 1)
    refined = refined_cm.reshape(1, c, m, 1)
    return out, refined


# speedup vs baseline: 1.3754x; 1.2155x over previous
import jax
import jax.numpy as jnp
from jax.experimental import pallas as pl
from jax.experimental.pallas import tpu as pltpu

K_DPC = 5
TOPK = 5
TN = 1792

HIGH = jax.lax.Precision.HIGHEST
LOW = jax.lax.Precision.DEFAULT


def _split(a):
    hi = a.astype(jnp.bfloat16)
    lo = (a - hi.astype(jnp.float32)).astype(jnp.bfloat16)
    return hi, lo


def _dot3(a, b, dims):
    ahi, alo = _split(a)
    bhi, blo = _split(b)
    d = lambda u, v: jax.lax.dot_general(u, v, dims,
                                         preferred_element_type=jnp.float32)
    return d(ahi, bhi) + (d(ahi, blo) + d(alo, bhi))


def _dpc_kernel(tok_ref, rp_ref, cent_ref):
    tok = jnp.transpose(tok_ref[...])
    npts = tok.shape[0]
    aa = jnp.sum(tok * tok, axis=1, keepdims=True)
    ab = jax.lax.dot_general(tok, tok, (((1,), (1,)), ((), ())),
                             precision=LOW)
    d2 = jnp.maximum(aa + jnp.transpose(aa) - 2.0 * ab, 0.0)
    dist = jnp.sqrt(d2 + 1e-12) + rp_ref[...]
    colid = jax.lax.broadcasted_iota(jnp.int32, (npts, npts), 1)
    work = dist
    acc = jnp.zeros((npts, 1), jnp.float32)
    for _ in range(K_DPC):
        mn = jnp.min(work, axis=1, keepdims=True)
        sel = jnp.min(jnp.where(work == mn, colid, npts), axis=1, keepdims=True)
        acc = acc + mn * mn
        work = jnp.where(colid == sel, jnp.inf, work)
    density = jnp.exp(-(acc / jnp.float32(K_DPC)))
    higher = jnp.transpose(density) > density
    dist_max = jnp.max(dist)
    delta = jnp.min(jnp.where(higher, dist, dist_max), axis=1, keepdims=True)
    score = delta * density
    st = jnp.transpose(score)
    rowid = jax.lax.broadcasted_iota(jnp.int32, (npts, npts), 0)
    rank = (jnp.sum((st > score).astype(jnp.int32), axis=1, keepdims=True)
            + jnp.sum(((st == score) & (colid < rowid)).astype(jnp.int32),
                      axis=1, keepdims=True))
    m = cent_ref.shape[0]
    sel_mat = (jax.lax.broadcasted_iota(jnp.int32, (m, npts), 0)
               == jnp.transpose(rank)).astype(jnp.float32)
    cent_ref[...] = jax.lax.dot_general(sel_mat, tok, (((1,), (0,)), ((), ())),
                                        precision=HIGH)


def _pass1_kernel(cent_ref, x_ref, idx_ref, num_ref, den_ref):
    i = pl.program_id(0)
    cent = cent_ref[...]
    xt = x_ref[...]
    m, c = cent.shape
    sim = jax.lax.dot_general(cent, xt, (((1,), (0,)), ((), ())),
                              precision=LOW) / jnp.sqrt(jnp.float32(c))
    e = jnp.exp(sim - jnp.max(sim, axis=0, keepdims=True))
    assign = e / jnp.sum(e, axis=0, keepdims=True)

    @pl.when(i == 0)
    def _():
        num_ref[...] = jnp.zeros_like(num_ref)
        den_ref[...] = jnp.zeros_like(den_ref)

    num_ref[...] += jax.lax.dot_general(assign, xt, (((1,), (1,)), ((), ())),
                                        precision=LOW)
    den_ref[...] += jnp.sum(assign, axis=1, keepdims=True)

    rowid = jax.lax.broadcasted_iota(jnp.int32, assign.shape, 0)
    work = assign
    rows = []
    for _ in range(TOPK):
        mx = jnp.max(work, axis=0, keepdims=True)
        sel = jnp.min(jnp.where(work == mx, rowid, m), axis=0, keepdims=True)
        rows.append(sel)
        work = jnp.where(rowid == sel, -jnp.inf, work)
    pad = jnp.zeros_like(rows[0])
    idx_ref[...] = jnp.concatenate(rows + [pad, pad, pad], axis=0)


def _prep_kernel(num_ref, den_ref, W1_ref, b1_ref, W2_ref, b2_ref, Wnn_ref,
                 refined_ref, rhi_ref, rlo_ref, Wa_ref, Wb_ref):
    agg = num_ref[...] / (den_ref[...] + 1e-6)
    h1 = jax.lax.dot_general(agg, W1_ref[...], (((1,), (1,)), ((), ())),
                             precision=LOW) + b1_ref[...]
    h1 = jnp.maximum(h1, 0.0)
    ffn = jax.lax.dot_general(h1, W2_ref[...], (((1,), (1,)), ((), ())),
                              precision=LOW) + b2_ref[...]
    refined = agg + ffn
    m = refined.shape[0]
    eye_m = (jax.lax.broadcasted_iota(jnp.int32, (m, m), 0)
             == jax.lax.broadcasted_iota(jnp.int32, (m, m), 1)).astype(jnp.float32)
    refined_ref[...] = jax.lax.dot_general(refined, eye_m, (((0,), (0,)), ((), ())),
                                           precision=HIGH)
    rhi = refined.astype(jnp.bfloat16)
    rhi_ref[...] = rhi
    rlo_ref[...] = (refined - rhi.astype(jnp.float32)).astype(jnp.bfloat16)
    Wnn = Wnn_ref[...]
    c = Wnn.shape[0]
    r2 = jax.lax.broadcasted_iota(jnp.int32, (2 * c, c), 0)
    c2 = jax.lax.broadcasted_iota(jnp.int32, (2 * c, c), 1)
    sa = (r2 == 2 * c2).astype(jnp.float32)
    sb = (r2 == 2 * c2 + 1).astype(jnp.float32)
    Wa_ref[...] = jax.lax.dot_general(Wnn, sa, (((1,), (0,)), ((), ())),
                                      precision=HIGH)
    Wb_ref[...] = jax.lax.dot_general(Wnn, sb, (((1,), (0,)), ((), ())),
                                      precision=HIGH)


def _pass2_kernel(x_ref, idx_ref, rhi_ref, rlo_ref, Wa_ref, Wb_ref, bnn_ref,
                  z_ref, stats_ref, s_ref, s2_ref):
    i = pl.program_id(0)
    xt = x_ref[...]
    rhi = rhi_ref[...]
    rlo = rlo_ref[...]
    m = rhi.shape[0]
    idx = idx_ref[...]
    rowid = jax.lax.broadcasted_iota(jnp.int32, (m, xt.shape[1]), 0)
    mx = None
    for k in range(TOPK):
        hot = (rowid == idx[k:k + 1, :]).astype(jnp.bfloat16)
        g = jax.lax.dot_general(
            rhi, hot, (((0,), (0,)), ((), ())),
            preferred_element_type=jnp.float32)
        g = g + jax.lax.dot_general(
            rlo, hot, (((0,), (0,)), ((), ())),
            preferred_element_type=jnp.float32)
        mx = g if mx is None else jnp.maximum(mx, g)
    xj = mx - xt
    z = (jax.lax.dot_general(Wa_ref[...], xt, (((1,), (0,)), ((), ())),
                             precision=LOW)
         + jax.lax.dot_general(Wb_ref[...], xj, (((1,), (0,)), ((), ())),
                               precision=LOW)
         + bnn_ref[...])
    z_ref[...] = z

    @pl.when(i == 0)
    def _():
        s_ref[...] = jnp.zeros_like(s_ref)
        s2_ref[...] = jnp.zeros_like(s2_ref)

    s_ref[...] += jnp.sum(z, axis=1, keepdims=True)
    s2_ref[...] += jnp.sum(z * z, axis=1, keepdims=True)

    @pl.when(i == pl.num_programs(0) - 1)
    def _():
        stats_ref[...] = jnp.concatenate(
            [s_ref[...], s2_ref[...],
             jnp.zeros((s_ref.shape[0], 6), jnp.float32)], axis=1)


def _bn_kernel(z_ref, stats_ref, gamma_ref, beta_ref, out_ref):
    n = jnp.float32(z_ref.shape[1] * pl.num_programs(0))
    mu = stats_ref[:, 0:1] / n
    var = stats_ref[:, 1:2] / n - mu * mu
    inv = gamma_ref[...] / jnp.sqrt(var + 1e-5)
    out_ref[...] = jnp.maximum((z_ref[...] - mu) * inv + beta_ref[...], 0.0)


def kernel(x, relative_pos, y, W1, b1, W2, b2, Wnn, bnn, gamma, beta):
    b, c, h, w = x.shape
    n = h * w
    x_flat = x.reshape(c, n)
    npts = y.shape[2]
    m = int(0.25 * npts)
    tok = y[0, :, :, 0]

    centroids = pl.pallas_call(
        _dpc_kernel,
        out_shape=jax.ShapeDtypeStruct((m, c), jnp.float32),
    )(tok, relative_pos[0])

    nsteps = n // TN
    idx, num, den = pl.pallas_call(
        _pass1_kernel,
        grid=(nsteps,),
        in_specs=[
            pl.BlockSpec((m, c), lambda i: (0, 0)),
            pl.BlockSpec((c, TN), lambda i: (0, i)),
        ],
        out_specs=[
            pl.BlockSpec((8, TN), lambda i: (0, i)),
            pl.BlockSpec((m, c), lambda i: (0, 0)),
            pl.BlockSpec((m, 1), lambda i: (0, 0)),
        ],
        out_shape=[
            jax.ShapeDtypeStruct((8, n), jnp.int32),
            jax.ShapeDtypeStruct((m, c), jnp.float32),
            jax.ShapeDtypeStruct((m, 1), jnp.float32),
        ],
        compiler_params=pltpu.CompilerParams(
            dimension_semantics=("arbitrary",)),
    )(centroids, x_flat)

    refined_cm, rhi, rlo, Wa, Wb = pl.pallas_call(
        _prep_kernel,
        out_shape=[
            jax.ShapeDtypeStruct((c, m), jnp.float32),
            jax.ShapeDtypeStruct((m, c), jnp.bfloat16),
            jax.ShapeDtypeStruct((m, c), jnp.bfloat16),
            jax.ShapeDtypeStruct((c, c), jnp.float32),
            jax.ShapeDtypeStruct((c, c), jnp.float32),
        ],
    )(num, den, W1, b1.reshape(1, -1), W2, b2.reshape(1, -1), Wnn)

    z, stats = pl.pallas_call(
        _pass2_kernel,
        grid=(nsteps,),
        in_specs=[
            pl.BlockSpec((c, TN), lambda i: (0, i)),
            pl.BlockSpec((8, TN), lambda i: (0, i)),
            pl.BlockSpec((m, c), lambda i: (0, 0)),
            pl.BlockSpec((m, c), lambda i: (0, 0)),
            pl.BlockSpec((c, c), lambda i: (0, 0)),
            pl.BlockSpec((c, c), lambda i: (0, 0)),
            pl.BlockSpec((c, 1), lambda i: (0, 0)),
        ],
        out_specs=[
            pl.BlockSpec((c, TN), lambda i: (0, i)),
            pl.BlockSpec((c, 8), lambda i: (0, 0)),
        ],
        out_shape=[
            jax.ShapeDtypeStruct((c, n), jnp.float32),
            jax.ShapeDtypeStruct((c, 8), jnp.float32),
        ],
        scratch_shapes=[
            pltpu.VMEM((c, 1), jnp.float32),
            pltpu.VMEM((c, 1), jnp.float32),
        ],
        compiler_params=pltpu.CompilerParams(
            dimension_semantics=("arbitrary",)),
    )(x_flat, idx, rhi, rlo, Wa, Wb, bnn.reshape(-1, 1))

    out2d = pl.pallas_call(
        _bn_kernel,
        grid=(nsteps,),
        in_specs=[
            pl.BlockSpec((c, TN), lambda i: (0, i)),
            pl.BlockSpec((c, 8), lambda i: (0, 0)),
            pl.BlockSpec((c, 1), lambda i: (0, 0)),
            pl.BlockSpec((c, 1), lambda i: (0, 0)),
        ],
        out_specs=pl.BlockSpec((c, TN), lambda i: (0, i)),
        out_shape=jax.ShapeDtypeStruct((c, n), jnp.float32),
        compiler_params=pltpu.CompilerParams(
            dimension_semantics=("arbitrary",)),
    )(z, stats, gamma.reshape(-1, 1), beta.reshape(-1, 1))

    out = out2d.reshape(1, c, n, 1)
    refined = refined_cm.reshape(1, c, m, 1)
    return out, refined


# speedup vs baseline: 1.4561x; 1.0586x over previous
import jax
import jax.numpy as jnp
from jax.experimental import pallas as pl
from jax.experimental.pallas import tpu as pltpu

K_DPC = 5
TOPK = 5
TN = 3584

HIGH = jax.lax.Precision.HIGHEST
LOW = jax.lax.Precision.DEFAULT


def _split(a):
    hi = a.astype(jnp.bfloat16)
    lo = (a - hi.astype(jnp.float32)).astype(jnp.bfloat16)
    return hi, lo


def _dot3(a, b, dims):
    ahi, alo = _split(a)
    bhi, blo = _split(b)
    d = lambda u, v: jax.lax.dot_general(u, v, dims,
                                         preferred_element_type=jnp.float32)
    return d(ahi, bhi) + (d(ahi, blo) + d(alo, bhi))


def _dpc_kernel(tok_ref, rp_ref, cent_ref):
    tok = jnp.transpose(tok_ref[...])
    npts = tok.shape[0]
    aa = jnp.sum(tok * tok, axis=1, keepdims=True)
    ab = jax.lax.dot_general(tok, tok, (((1,), (1,)), ((), ())),
                             precision=LOW)
    d2 = jnp.maximum(aa + jnp.transpose(aa) - 2.0 * ab, 0.0)
    dist = jnp.sqrt(d2 + 1e-12) + rp_ref[...]
    colid = jax.lax.broadcasted_iota(jnp.int32, (npts, npts), 1)
    work = dist
    acc = jnp.zeros((npts, 1), jnp.float32)
    for _ in range(K_DPC):
        mn = jnp.min(work, axis=1, keepdims=True)
        sel = jnp.min(jnp.where(work == mn, colid, npts), axis=1, keepdims=True)
        acc = acc + mn * mn
        work = jnp.where(colid == sel, jnp.inf, work)
    density = jnp.exp(-(acc / jnp.float32(K_DPC)))
    higher = jnp.transpose(density) > density
    dist_max = jnp.max(dist)
    delta = jnp.min(jnp.where(higher, dist, dist_max), axis=1, keepdims=True)
    score = delta * density
    st = jnp.transpose(score)
    rowid = jax.lax.broadcasted_iota(jnp.int32, (npts, npts), 0)
    rank = (jnp.sum((st > score).astype(jnp.int32), axis=1, keepdims=True)
            + jnp.sum(((st == score) & (colid < rowid)).astype(jnp.int32),
                      axis=1, keepdims=True))
    m = cent_ref.shape[0]
    sel_mat = (jax.lax.broadcasted_iota(jnp.int32, (m, npts), 0)
               == jnp.transpose(rank)).astype(jnp.float32)
    cent_ref[...] = jax.lax.dot_general(sel_mat, tok, (((1,), (0,)), ((), ())),
                                        precision=HIGH)


def _pass1_kernel(cent_ref, x_ref, idx_ref, num_ref, den_ref):
    i = pl.program_id(0)
    cent = cent_ref[...]
    xt = x_ref[...]
    m, c = cent.shape
    sim = jax.lax.dot_general(cent, xt, (((1,), (0,)), ((), ())),
                              precision=LOW) / jnp.sqrt(jnp.float32(c))
    e = jnp.exp(sim - jnp.max(sim, axis=0, keepdims=True))
    assign = e / jnp.sum(e, axis=0, keepdims=True)

    @pl.when(i == 0)
    def _():
        num_ref[...] = jnp.zeros_like(num_ref)
        den_ref[...] = jnp.zeros_like(den_ref)

    num_ref[...] += jax.lax.dot_general(assign, xt, (((1,), (1,)), ((), ())),
                                        precision=LOW)
    den_ref[...] += jnp.sum(assign, axis=1, keepdims=True)

    rowid = jax.lax.broadcasted_iota(jnp.int32, assign.shape, 0)
    work = assign
    rows = []
    for _ in range(TOPK):
        mx = jnp.max(work, axis=0, keepdims=True)
        sel = jnp.min(jnp.where(work == mx, rowid, m), axis=0, keepdims=True)
        rows.append(sel)
        work = jnp.where(rowid == sel, -jnp.inf, work)
    pad = jnp.zeros_like(rows[0])
    idx_ref[...] = jnp.concatenate(rows + [pad, pad, pad], axis=0)


def _prep_kernel(num_ref, den_ref, W1_ref, b1_ref, W2_ref, b2_ref, Wnn_ref,
                 refined_ref, rhi_ref, rlo_ref, Wa_ref, Wb_ref):
    agg = num_ref[...] / (den_ref[...] + 1e-6)
    h1 = jax.lax.dot_general(agg, W1_ref[...], (((1,), (1,)), ((), ())),
                             precision=LOW) + b1_ref[...]
    h1 = jnp.maximum(h1, 0.0)
    ffn = jax.lax.dot_general(h1, W2_ref[...], (((1,), (1,)), ((), ())),
                              precision=LOW) + b2_ref[...]
    refined = agg + ffn
    m = refined.shape[0]
    eye_m = (jax.lax.broadcasted_iota(jnp.int32, (m, m), 0)
             == jax.lax.broadcasted_iota(jnp.int32, (m, m), 1)).astype(jnp.float32)
    refined_ref[...] = jax.lax.dot_general(refined, eye_m, (((0,), (0,)), ((), ())),
                                           precision=HIGH)
    rhi = refined.astype(jnp.bfloat16)
    rhi_ref[...] = rhi
    rlo_ref[...] = (refined - rhi.astype(jnp.float32)).astype(jnp.bfloat16)
    Wnn = Wnn_ref[...]
    c = Wnn.shape[0]
    r2 = jax.lax.broadcasted_iota(jnp.int32, (2 * c, c), 0)
    c2 = jax.lax.broadcasted_iota(jnp.int32, (2 * c, c), 1)
    sa = (r2 == 2 * c2).astype(jnp.float32)
    sb = (r2 == 2 * c2 + 1).astype(jnp.float32)
    Wa_ref[...] = jax.lax.dot_general(Wnn, sa, (((1,), (0,)), ((), ())),
                                      precision=HIGH)
    Wb_ref[...] = jax.lax.dot_general(Wnn, sb, (((1,), (0,)), ((), ())),
                                      precision=HIGH)


def _pass2_kernel(x_ref, idx_ref, rhi_ref, rlo_ref, Wa_ref, Wb_ref, bnn_ref,
                  z_ref, stats_ref, s_ref, s2_ref):
    i = pl.program_id(0)
    xt = x_ref[...]
    rhi = rhi_ref[...]
    rlo = rlo_ref[...]
    m = rhi.shape[0]
    idx = idx_ref[...]
    rowid = jax.lax.broadcasted_iota(jnp.int32, (m, xt.shape[1]), 0)
    mx = None
    for k in range(TOPK):
        hot = (rowid == idx[k:k + 1, :]).astype(jnp.bfloat16)
        g = jax.lax.dot_general(
            rhi, hot, (((0,), (0,)), ((), ())),
            preferred_element_type=jnp.float32)
        g = g + jax.lax.dot_general(
            rlo, hot, (((0,), (0,)), ((), ())),
            preferred_element_type=jnp.float32)
        mx = g if mx is None else jnp.maximum(mx, g)
    xj = mx - xt
    z = (jax.lax.dot_general(Wa_ref[...], xt, (((1,), (0,)), ((), ())),
                             precision=LOW)
         + jax.lax.dot_general(Wb_ref[...], xj, (((1,), (0,)), ((), ())),
                               precision=LOW)
         + bnn_ref[...])
    z_ref[...] = z

    @pl.when(i == 0)
    def _():
        s_ref[...] = jnp.zeros_like(s_ref)
        s2_ref[...] = jnp.zeros_like(s2_ref)

    s_ref[...] += jnp.sum(z, axis=1, keepdims=True)
    s2_ref[...] += jnp.sum(z * z, axis=1, keepdims=True)

    @pl.when(i == pl.num_programs(0) - 1)
    def _():
        stats_ref[...] = jnp.concatenate(
            [s_ref[...], s2_ref[...],
             jnp.zeros((s_ref.shape[0], 6), jnp.float32)], axis=1)


def _bn_kernel(z_ref, stats_ref, gamma_ref, beta_ref, out_ref):
    n = jnp.float32(z_ref.shape[1] * pl.num_programs(0))
    mu = stats_ref[:, 0:1] / n
    var = stats_ref[:, 1:2] / n - mu * mu
    inv = gamma_ref[...] / jnp.sqrt(var + 1e-5)
    out_ref[...] = jnp.maximum((z_ref[...] - mu) * inv + beta_ref[...], 0.0)


def kernel(x, relative_pos, y, W1, b1, W2, b2, Wnn, bnn, gamma, beta):
    b, c, h, w = x.shape
    n = h * w
    x_flat = x.reshape(c, n)
    npts = y.shape[2]
    m = int(0.25 * npts)
    tok = y[0, :, :, 0]

    centroids = pl.pallas_call(
        _dpc_kernel,
        out_shape=jax.ShapeDtypeStruct((m, c), jnp.float32),
    )(tok, relative_pos[0])

    nsteps = n // TN
    idx, num, den = pl.pallas_call(
        _pass1_kernel,
        grid=(nsteps,),
        in_specs=[
            pl.BlockSpec((m, c), lambda i: (0, 0)),
            pl.BlockSpec((c, TN), lambda i: (0, i)),
        ],
        out_specs=[
            pl.BlockSpec((8, TN), lambda i: (0, i)),
            pl.BlockSpec((m, c), lambda i: (0, 0)),
            pl.BlockSpec((m, 1), lambda i: (0, 0)),
        ],
        out_shape=[
            jax.ShapeDtypeStruct((8, n), jnp.int32),
            jax.ShapeDtypeStruct((m, c), jnp.float32),
            jax.ShapeDtypeStruct((m, 1), jnp.float32),
        ],
        compiler_params=pltpu.CompilerParams(
            dimension_semantics=("arbitrary",)),
    )(centroids, x_flat)

    refined_cm, rhi, rlo, Wa, Wb = pl.pallas_call(
        _prep_kernel,
        out_shape=[
            jax.ShapeDtypeStruct((c, m), jnp.float32),
            jax.ShapeDtypeStruct((m, c), jnp.bfloat16),
            jax.ShapeDtypeStruct((m, c), jnp.bfloat16),
            jax.ShapeDtypeStruct((c, c), jnp.float32),
            jax.ShapeDtypeStruct((c, c), jnp.float32),
        ],
    )(num, den, W1, b1.reshape(1, -1), W2, b2.reshape(1, -1), Wnn)

    z, stats = pl.pallas_call(
        _pass2_kernel,
        grid=(nsteps,),
        in_specs=[
            pl.BlockSpec((c, TN), lambda i: (0, i)),
            pl.BlockSpec((8, TN), lambda i: (0, i)),
            pl.BlockSpec((m, c), lambda i: (0, 0)),
            pl.BlockSpec((m, c), lambda i: (0, 0)),
            pl.BlockSpec((c, c), lambda i: (0, 0)),
            pl.BlockSpec((c, c), lambda i: (0, 0)),
            pl.BlockSpec((c, 1), lambda i: (0, 0)),
        ],
        out_specs=[
            pl.BlockSpec((c, TN), lambda i: (0, i)),
            pl.BlockSpec((c, 8), lambda i: (0, 0)),
        ],
        out_shape=[
            jax.ShapeDtypeStruct((c, n), jnp.float32),
            jax.ShapeDtypeStruct((c, 8), jnp.float32),
        ],
        scratch_shapes=[
            pltpu.VMEM((c, 1), jnp.float32),
            pltpu.VMEM((c, 1), jnp.float32),
        ],
        compiler_params=pltpu.CompilerParams(
            dimension_semantics=("arbitrary",)),
    )(x_flat, idx, rhi, rlo, Wa, Wb, bnn.reshape(-1, 1))

    out2d = pl.pallas_call(
        _bn_kernel,
        grid=(nsteps,),
        in_specs=[
            pl.BlockSpec((c, TN), lambda i: (0, i)),
            pl.BlockSpec((c, 8), lambda i: (0, 0)),
            pl.BlockSpec((c, 1), lambda i: (0, 0)),
            pl.BlockSpec((c, 1), lambda i: (0, 0)),
        ],
        out_specs=pl.BlockSpec((c, TN), lambda i: (0, i)),
        out_shape=jax.ShapeDtypeStruct((c, n), jnp.float32),
        compiler_params=pltpu.CompilerParams(
            dimension_semantics=("arbitrary",)),
    )(z, stats, gamma.reshape(-1, 1), beta.reshape(-1, 1))

    out = out2d.reshape(1, c, n, 1)
    refined = refined_cm.reshape(1, c, m, 1)
    return out, refined
